# baseline (speedup 1.0000x reference)
import functools

import jax
import jax.numpy as jnp
from jax import lax
from jax.experimental import pallas as pl
from jax.experimental.pallas import tpu as pltpu
from jax.experimental.pallas import tpu_sc as plsc

HIDDEN = 128
N_NODES = 10000
N_EDGES = 320000
D_MAX = 6.0
D_COUNT = 16
MU_STEP = D_MAX / (D_COUNT - 1)
INV_SIGMA = D_COUNT / D_MAX

_NC = 2
_NS = 16
_NW = _NC * _NS
_LANES = 16

_SEG = 5
_SEG_E = N_EDGES // _SEG
_SEG_EPW = _SEG_E // _NW
_ROWS = 16000
_BLKS = _SEG_E // _ROWS


def _sc_d2_body(px_hbm, py_hbm, pz_hbm, src_hbm, dst_hbm, out_hbm,
                px, py, pz, si, di, o):
    wid = lax.axis_index("s") * _NC + lax.axis_index("c")
    base = wid * _SEG_EPW
    pltpu.sync_copy(px_hbm, px)
    pltpu.sync_copy(py_hbm, py)
    pltpu.sync_copy(pz_hbm, pz)
    pltpu.sync_copy(src_hbm.at[pl.ds(base, _SEG_EPW)], si)
    pltpu.sync_copy(dst_hbm.at[pl.ds(base, _SEG_EPW)], di)

    def body(i, _):
        s = si[pl.ds(i * _LANES, _LANES)]
        d = di[pl.ds(i * _LANES, _LANES)]
        dx = plsc.load_gather(px, [s]) - plsc.load_gather(px, [d])
        dy = plsc.load_gather(py, [s]) - plsc.load_gather(py, [d])
        dz = plsc.load_gather(pz, [s]) - plsc.load_gather(pz, [d])
        o[pl.ds(i * _LANES, _LANES)] = dx * dx + dy * dy + dz * dz
        return 0

    lax.fori_loop(0, _SEG_EPW // _LANES, body, 0)
    pltpu.sync_copy(o, out_hbm.at[pl.ds(base, _SEG_EPW)])


@functools.lru_cache(maxsize=1)
def _make_sc_d2():
    return functools.partial(
        pl.kernel,
        mesh=plsc.VectorSubcoreMesh(core_axis_name="c", subcore_axis_name="s",
                                    num_cores=_NC, num_subcores=_NS),
        out_type=jax.ShapeDtypeStruct((_SEG_E,), jnp.float32),
        compiler_params=pltpu.CompilerParams(needs_layout_passes=False),
        scratch_types=[
            pltpu.VMEM((N_NODES,), jnp.float32),
            pltpu.VMEM((N_NODES,), jnp.float32),
            pltpu.VMEM((N_NODES,), jnp.float32),
            pltpu.VMEM((_SEG_EPW,), jnp.int32),
            pltpu.VMEM((_SEG_EPW,), jnp.int32),
            pltpu.VMEM((_SEG_EPW,), jnp.float32),
        ],
    )(_sc_d2_body)


def _rbf_mlp(d2_ref, w_ref, b_ref):
    mu = (lax.broadcasted_iota(jnp.int32, (1, D_COUNT), 1)
          .astype(jnp.float32) * MU_STEP)
    dist = jnp.sqrt(d2_ref[...])
    t = (dist - mu) * INV_SIGMA
    rbf = jnp.exp(-(t * t))
    z = jnp.dot(rbf, w_ref[...], preferred_element_type=jnp.float32)
    return jax.nn.sigmoid(z + b_ref[...])


def _tc_body_first(d2_ref, w_ref, b_ref, out_ref):
    out_ref[...] = _rbf_mlp(d2_ref, w_ref, b_ref)


def _tc_body_chain(d2_ref, w_ref, b_ref, prev_ref, out_ref):
    del prev_ref
    out_ref[...] = _rbf_mlp(d2_ref, w_ref, b_ref)


def _tc_mlp_seg(k, d2_seg, W, b2, out):
    d2_spec = pl.BlockSpec((_ROWS, 1), lambda i: (i, 0))
    w_spec = pl.BlockSpec((D_COUNT, HIDDEN), lambda i: (0, 0))
    b_spec = pl.BlockSpec((1, HIDDEN), lambda i: (0, 0))
    out_spec = pl.BlockSpec((_ROWS, HIDDEN), lambda i, k=k: (i + k * _BLKS, 0))
    out_shape = jax.ShapeDtypeStruct((N_EDGES, HIDDEN), jnp.float32)
    d2_seg = d2_seg.reshape(_SEG_E, 1)
    if k == 0:
        return pl.pallas_call(
            _tc_body_first,
            grid=(_BLKS,),
            in_specs=[d2_spec, w_spec, b_spec],
            out_specs=out_spec,
            out_shape=out_shape,
        )(d2_seg, W, b2)
    return pl.pallas_call(
        _tc_body_chain,
        grid=(_BLKS,),
        in_specs=[d2_spec, w_spec, b_spec,
                  pl.BlockSpec(memory_space=pl.ANY)],
        out_specs=out_spec,
        out_shape=out_shape,
        input_output_aliases={3: 0},
    )(d2_seg, W, b2, out)


def kernel(pos, edge_index, W, b):
    pos = pos.astype(jnp.float32)
    ei = edge_index.astype(jnp.int32)
    pt = pos.T
    px, py, pz = pt[0], pt[1], pt[2]
    sc = _make_sc_d2()
    d2s = [sc(px, py, pz,
              lax.dynamic_slice_in_dim(ei[0], k * _SEG_E, _SEG_E),
              lax.dynamic_slice_in_dim(ei[1], k * _SEG_E, _SEG_E))
           for k in range(_SEG)]
    b2 = b.reshape(1, HIDDEN)
    out = None
    for k in range(_SEG):
        out = _tc_mlp_seg(k, d2s[k], W, b2, out)
    return out


# speedup vs baseline: 1.1015x; 1.1015x over previous
import functools

import jax
import jax.numpy as jnp
from jax import lax
from jax.experimental import pallas as pl
from jax.experimental.pallas import tpu as pltpu
from jax.experimental.pallas import tpu_sc as plsc

HIDDEN = 128
N_NODES = 10000
N_EDGES = 320000
D_MAX = 6.0
D_COUNT = 16
MU_STEP = D_MAX / (D_COUNT - 1)
INV_SIGMA = D_COUNT / D_MAX

_NC = 2
_NS = 16
_NW = _NC * _NS
_EPW = N_EDGES // _NW
_LANES = 16


def _sc_d2_body(px_hbm, py_hbm, pz_hbm, src_hbm, dst_hbm, out_hbm,
                px, py, pz, si, di, o):
    wid = lax.axis_index("s") * _NC + lax.axis_index("c")
    base = wid * _EPW
    pltpu.sync_copy(px_hbm, px)
    pltpu.sync_copy(py_hbm, py)
    pltpu.sync_copy(pz_hbm, pz)
    pltpu.sync_copy(src_hbm.at[pl.ds(base, _EPW)], si)
    pltpu.sync_copy(dst_hbm.at[pl.ds(base, _EPW)], di)

    def body(i, _):
        s = si[pl.ds(i * _LANES, _LANES)]
        d = di[pl.ds(i * _LANES, _LANES)]
        dx = plsc.load_gather(px, [s]) - plsc.load_gather(px, [d])
        dy = plsc.load_gather(py, [s]) - plsc.load_gather(py, [d])
        dz = plsc.load_gather(pz, [s]) - plsc.load_gather(pz, [d])
        o[pl.ds(i * _LANES, _LANES)] = dx * dx + dy * dy + dz * dz
        return 0

    lax.fori_loop(0, _EPW // _LANES, body, 0)
    pltpu.sync_copy(o, out_hbm.at[pl.ds(base, _EPW)])


@functools.lru_cache(maxsize=1)
def _make_sc_d2():
    return functools.partial(
        pl.kernel,
        mesh=plsc.VectorSubcoreMesh(core_axis_name="c", subcore_axis_name="s",
                                    num_cores=_NC, num_subcores=_NS),
        out_type=jax.ShapeDtypeStruct((N_EDGES,), jnp.float32),
        compiler_params=pltpu.CompilerParams(needs_layout_passes=False),
        scratch_types=[
            pltpu.VMEM((N_NODES,), jnp.float32),
            pltpu.VMEM((N_NODES,), jnp.float32),
            pltpu.VMEM((N_NODES,), jnp.float32),
            pltpu.VMEM((_EPW,), jnp.int32),
            pltpu.VMEM((_EPW,), jnp.int32),
            pltpu.VMEM((_EPW,), jnp.float32),
        ],
    )(_sc_d2_body)


_ROWS = 16000
_NBUF = 4


def _tc_body(d2_ref, w_ref, b_ref, out_ref, buf, sems):
    c = pl.program_id(0)
    nsteps = pl.num_programs(0)
    slot = lax.rem(c, _NBUF)

    @pl.when(c >= _NBUF)
    def _():
        pltpu.make_async_copy(
            buf.at[slot],
            out_ref.at[pl.ds((c - _NBUF) * _ROWS, _ROWS), :],
            sems.at[slot],
        ).wait()

    mu = (lax.broadcasted_iota(jnp.int32, (1, D_COUNT), 1)
          .astype(jnp.float32) * MU_STEP)
    dist = jnp.sqrt(d2_ref[...])
    t = (dist - mu) * INV_SIGMA
    rbf = jnp.exp(-(t * t))
    z = jnp.dot(rbf, w_ref[...], preferred_element_type=jnp.float32)
    buf[slot] = jax.nn.sigmoid(z + b_ref[...])
    pltpu.make_async_copy(
        buf.at[slot],
        out_ref.at[pl.ds(c * _ROWS, _ROWS), :],
        sems.at[slot],
    ).start()

    @pl.when(c == nsteps - 1)
    def _():
        for k in range(_NBUF):
            pltpu.make_async_copy(
                buf.at[k],
                out_ref.at[pl.ds(k * _ROWS, _ROWS), :],
                sems.at[k],
            ).wait()


def _tc_mlp(d2, W, b):
    return pl.pallas_call(
        _tc_body,
        grid=(N_EDGES // _ROWS,),
        in_specs=[
            pl.BlockSpec((_ROWS, 1), lambda i: (i, 0)),
            pl.BlockSpec((D_COUNT, HIDDEN), lambda i: (0, 0)),
            pl.BlockSpec((1, HIDDEN), lambda i: (0, 0)),
        ],
        out_specs=pl.BlockSpec(memory_space=pl.ANY),
        out_shape=jax.ShapeDtypeStruct((N_EDGES, HIDDEN), jnp.float32),
        scratch_shapes=[
            pltpu.VMEM((_NBUF, _ROWS, HIDDEN), jnp.float32),
            pltpu.SemaphoreType.DMA((_NBUF,)),
        ],
    )(d2.reshape(N_EDGES, 1), W, b.reshape(1, HIDDEN))


def kernel(pos, edge_index, W, b):
    pos = pos.astype(jnp.float32)
    ei = edge_index.astype(jnp.int32)
    pt = pos.T
    px, py, pz = pt[0], pt[1], pt[2]
    d2 = _make_sc_d2()(px, py, pz, ei[0], ei[1])
    return _tc_mlp(d2, W, b)


# speedup vs baseline: 1.1049x; 1.0030x over previous
import functools

import jax
import jax.numpy as jnp
from jax import lax
from jax.experimental import pallas as pl
from jax.experimental.pallas import tpu as pltpu
from jax.experimental.pallas import tpu_sc as plsc

HIDDEN = 128
N_NODES = 10000
N_EDGES = 320000
D_MAX = 6.0
D_COUNT = 16
MU_STEP = D_MAX / (D_COUNT - 1)
INV_SIGMA = D_COUNT / D_MAX

_NC = 2
_NS = 16
_NW = _NC * _NS
_EPW = N_EDGES // _NW
_LANES = 16
_UNROLL = 5


def _sc_d2_body(px_hbm, py_hbm, pz_hbm, src_hbm, dst_hbm, out_hbm,
                px, py, pz, si, di, o):
    wid = lax.axis_index("s") * _NC + lax.axis_index("c")
    base = wid * _EPW
    pltpu.sync_copy(px_hbm, px)
    pltpu.sync_copy(py_hbm, py)
    pltpu.sync_copy(pz_hbm, pz)
    pltpu.sync_copy(src_hbm.at[pl.ds(base, _EPW)], si)
    pltpu.sync_copy(dst_hbm.at[pl.ds(base, _EPW)], di)

    def body(i, _):
        for u in range(_UNROLL):
            off = (i * _UNROLL + u) * _LANES
            s = si[pl.ds(off, _LANES)]
            d = di[pl.ds(off, _LANES)]
            dx = plsc.load_gather(px, [s]) - plsc.load_gather(px, [d])
            dy = plsc.load_gather(py, [s]) - plsc.load_gather(py, [d])
            dz = plsc.load_gather(pz, [s]) - plsc.load_gather(pz, [d])
            o[pl.ds(off, _LANES)] = dx * dx + dy * dy + dz * dz
        return 0

    lax.fori_loop(0, _EPW // (_LANES * _UNROLL), body, 0)
    pltpu.sync_copy(o, out_hbm.at[pl.ds(base, _EPW)])


@functools.lru_cache(maxsize=1)
def _make_sc_d2():
    return functools.partial(
        pl.kernel,
        mesh=plsc.VectorSubcoreMesh(core_axis_name="c", subcore_axis_name="s",
                                    num_cores=_NC, num_subcores=_NS),
        out_type=jax.ShapeDtypeStruct((N_EDGES,), jnp.float32),
        compiler_params=pltpu.CompilerParams(needs_layout_passes=False),
        scratch_types=[
            pltpu.VMEM((N_NODES,), jnp.float32),
            pltpu.VMEM((N_NODES,), jnp.float32),
            pltpu.VMEM((N_NODES,), jnp.float32),
            pltpu.VMEM((_EPW,), jnp.int32),
            pltpu.VMEM((_EPW,), jnp.int32),
            pltpu.VMEM((_EPW,), jnp.float32),
        ],
    )(_sc_d2_body)


_ROWS = 20000


def _tc_body(d2_ref, w_ref, b_ref, out_ref):
    mu = (lax.broadcasted_iota(jnp.int32, (1, D_COUNT), 1)
          .astype(jnp.float32) * MU_STEP)
    dist = jnp.sqrt(d2_ref[...])
    t = (dist - mu) * INV_SIGMA
    rbf = jnp.exp(-(t * t))
    z = jnp.dot(rbf, w_ref[...], preferred_element_type=jnp.float32)
    out_ref[...] = jax.nn.sigmoid(z + b_ref[...])


def _tc_mlp(d2, W, b):
    return pl.pallas_call(
        _tc_body,
        grid=(N_EDGES // _ROWS,),
        in_specs=[
            pl.BlockSpec((_ROWS, 1), lambda i: (i, 0)),
            pl.BlockSpec((D_COUNT, HIDDEN), lambda i: (0, 0)),
            pl.BlockSpec((1, HIDDEN), lambda i: (0, 0)),
        ],
        out_specs=pl.BlockSpec((_ROWS, HIDDEN), lambda i: (i, 0)),
        out_shape=jax.ShapeDtypeStruct((N_EDGES, HIDDEN), jnp.float32),
    )(d2.reshape(N_EDGES, 1), W, b.reshape(1, HIDDEN))


def kernel(pos, edge_index, W, b):
    pos = pos.astype(jnp.float32)
    ei = edge_index.astype(jnp.int32)
    pt = pos.T
    px, py, pz = pt[0], pt[1], pt[2]
    d2 = _make_sc_d2()(px, py, pz, ei[0], ei[1])
    return _tc_mlp(d2, W, b)


# speedup vs baseline: 1.1134x; 1.0077x over previous
import functools

import jax
import jax.numpy as jnp
from jax import lax
from jax.experimental import pallas as pl
from jax.experimental.pallas import tpu as pltpu
from jax.experimental.pallas import tpu_sc as plsc

HIDDEN = 128
N_NODES = 10000
N_EDGES = 320000
D_MAX = 6.0
D_COUNT = 16
MU_STEP = D_MAX / (D_COUNT - 1)
INV_SIGMA = D_COUNT / D_MAX

_NC = 2
_NS = 16
_NW = _NC * _NS
_EPW = N_EDGES // _NW
_LANES = 16
_UNROLL = 5


def _sc_d2_body(px_hbm, py_hbm, pz_hbm, src_hbm, dst_hbm, out_hbm,
                px, py, pz, si, di, o, sems):
    wid = lax.axis_index("s") * _NC + lax.axis_index("c")
    base = wid * _EPW
    copies = [
        pltpu.make_async_copy(px_hbm, px, sems.at[0]),
        pltpu.make_async_copy(py_hbm, py, sems.at[1]),
        pltpu.make_async_copy(pz_hbm, pz, sems.at[2]),
        pltpu.make_async_copy(src_hbm.at[pl.ds(base, _EPW)], si, sems.at[3]),
        pltpu.make_async_copy(dst_hbm.at[pl.ds(base, _EPW)], di, sems.at[4]),
    ]
    for cp in copies:
        cp.start()
    for cp in copies:
        cp.wait()

    def body(i, _):
        for u in range(_UNROLL):
            off = (i * _UNROLL + u) * _LANES
            s = si[pl.ds(off, _LANES)]
            d = di[pl.ds(off, _LANES)]
            dx = plsc.load_gather(px, [s]) - plsc.load_gather(px, [d])
            dy = plsc.load_gather(py, [s]) - plsc.load_gather(py, [d])
            dz = plsc.load_gather(pz, [s]) - plsc.load_gather(pz, [d])
            o[pl.ds(off, _LANES)] = dx * dx + dy * dy + dz * dz
        return 0

    lax.fori_loop(0, _EPW // (_LANES * _UNROLL), body, 0)
    pltpu.sync_copy(o, out_hbm.at[pl.ds(base, _EPW)])


@functools.lru_cache(maxsize=1)
def _make_sc_d2():
    return functools.partial(
        pl.kernel,
        mesh=plsc.VectorSubcoreMesh(core_axis_name="c", subcore_axis_name="s",
                                    num_cores=_NC, num_subcores=_NS),
        out_type=jax.ShapeDtypeStruct((N_EDGES,), jnp.float32),
        compiler_params=pltpu.CompilerParams(needs_layout_passes=False),
        scratch_types=[
            pltpu.VMEM((N_NODES,), jnp.float32),
            pltpu.VMEM((N_NODES,), jnp.float32),
            pltpu.VMEM((N_NODES,), jnp.float32),
            pltpu.VMEM((_EPW,), jnp.int32),
            pltpu.VMEM((_EPW,), jnp.int32),
            pltpu.VMEM((_EPW,), jnp.float32),
            pltpu.SemaphoreType.DMA((5,)),
        ],
    )(_sc_d2_body)


_ROWS = 20000


def _tc_body(d2_ref, w_ref, b_ref, out_ref):
    mu = (lax.broadcasted_iota(jnp.int32, (1, D_COUNT), 1)
          .astype(jnp.float32) * MU_STEP)
    dist = jnp.sqrt(d2_ref[...])
    t = (dist - mu) * INV_SIGMA
    rbf = jnp.exp(-(t * t))
    z = jnp.dot(rbf, w_ref[...], preferred_element_type=jnp.float32)
    out_ref[...] = jax.nn.sigmoid(z + b_ref[...])


def _tc_mlp(d2, W, b):
    return pl.pallas_call(
        _tc_body,
        grid=(N_EDGES // _ROWS,),
        in_specs=[
            pl.BlockSpec((_ROWS, 1), lambda i: (i, 0)),
            pl.BlockSpec((D_COUNT, HIDDEN), lambda i: (0, 0)),
            pl.BlockSpec((1, HIDDEN), lambda i: (0, 0)),
        ],
        out_specs=pl.BlockSpec((_ROWS, HIDDEN), lambda i: (i, 0)),
        out_shape=jax.ShapeDtypeStruct((N_EDGES, HIDDEN), jnp.float32),
    )(d2.reshape(N_EDGES, 1), W, b.reshape(1, HIDDEN))


def kernel(pos, edge_index, W, b):
    pos = pos.astype(jnp.float32)
    ei = edge_index.astype(jnp.int32)
    pt = pos.T
    px, py, pz = pt[0], pt[1], pt[2]
    d2 = _make_sc_d2()(px, py, pz, ei[0], ei[1])
    return _tc_mlp(d2, W, b)
